# Initial kernel scaffold; baseline (speedup 1.0000x reference)
import jax
import jax.numpy as jnp
from jax.experimental import pallas as pl


def kernel(inputs, edge_index, edge_weight, Ws, bs):
    raise NotImplementedError("write your pallas kernel here")


# speedup vs baseline: 2.1626x; 2.1626x over previous
import jax
import jax.numpy as jnp
from jax.experimental import pallas as pl

_N = 10000
_D = 128


def _matmul_block(h_ref, w_ref, o_ref):
    o_ref[...] = jnp.dot(h_ref[...], w_ref[...],
                         preferred_element_type=jnp.float32)


def _pallas_matmul(h, W):
    return pl.pallas_call(
        _matmul_block,
        out_shape=jax.ShapeDtypeStruct((_N, _D), jnp.float32),
    )(h, W)


def kernel(inputs, edge_index, edge_weight, Ws, bs):
    src = edge_index[0]
    dst = edge_index[1]
    w = jnp.where(src == dst, jnp.zeros_like(edge_weight), edge_weight)
    deg_out = jax.ops.segment_sum(w, src, num_segments=_N) + 1.0
    deg_in = jax.ops.segment_sum(w, dst, num_segments=_N) + 1.0
    a = jax.lax.rsqrt(deg_out)
    b = jax.lax.rsqrt(deg_in)

    h = inputs
    for l in range(3):
        hw = _pallas_matmul(h, Ws[l])
        g = a[:, None] * hw
        msg = g[src] * w[:, None]
        s = jax.ops.segment_sum(msg, dst, num_segments=_N)
        h = b[:, None] * (s + g) + bs[l]
    return h


# speedup vs baseline: 7.7656x; 3.5909x over previous
import dataclasses
import functools

import jax
import jax.numpy as jnp
from jax import lax
from jax.experimental import pallas as pl
from jax.experimental.pallas import tpu as pltpu
from jax.experimental.pallas import tpu_sc as plsc

_N = 10000
_E = 320000
_D = 128
_NC = 2
_NS = 16
_NW = _NC * _NS
_EPW = _E // _NW
_B = 125
_C = _EPW // _B
_RQ = 624
_RL = _N - (_NS - 1) * _RQ

_mesh = plsc.VectorSubcoreMesh(core_axis_name="c", subcore_axis_name="s")

_cp = pltpu.CompilerParams()
if "needs_layout_passes" in pltpu.CompilerParams.__dataclass_fields__:
    _cp = dataclasses.replace(_cp, needs_layout_passes=False)


@functools.partial(
    pl.kernel,
    out_type=jax.ShapeDtypeStruct((_NC, _N, _D), jnp.float32),
    mesh=_mesh,
    compiler_params=_cp,
    scratch_types=[
        pltpu.VMEM((_C, _B), jnp.int32),
        pltpu.VMEM((_C, _B), jnp.int32),
        pltpu.VMEM((_C, _B), jnp.float32),
        pltpu.VMEM((_B, _D), jnp.float32),
        pltpu.VMEM_SHARED((_N, _D), jnp.float32),
    ],
)
def _sc_edge_pass(g_hbm, src_hbm, dst_hbm, w_hbm, zero_hbm, out_hbm,
                  src_v, dst_v, w_v, rows_v, acc_s):
    cid = lax.axis_index("c")
    sid = lax.axis_index("s")
    wid = sid * _NC + cid

    row0 = sid * _RQ

    @pl.when(sid < _NS - 1)
    def _():
        pltpu.sync_copy(zero_hbm.at[pl.ds(row0, _RQ)],
                        acc_s.at[pl.ds(row0, _RQ)])

    @pl.when(sid == _NS - 1)
    def _():
        pltpu.sync_copy(zero_hbm.at[pl.ds(row0, _RL)],
                        acc_s.at[pl.ds(row0, _RL)])

    pltpu.sync_copy(src_hbm.at[wid], src_v)
    pltpu.sync_copy(dst_hbm.at[wid], dst_v)
    pltpu.sync_copy(w_hbm.at[wid], w_v)

    plsc.subcore_barrier()

    @pl.loop(0, _C)
    def _(j):
        pltpu.sync_copy(g_hbm.at[src_v.at[j]], rows_v)

        @pl.loop(0, _B)
        def _(i):
            wsp = plsc.load_gather(
                w_v, [jnp.full((16,), j, jnp.int32),
                      jnp.full((16,), i, jnp.int32)])
            for c in range(_D // 16):
                sl = pl.ds(c * 16, 16)
                rows_v[i, sl] = rows_v[i, sl] * wsp

        pltpu.sync_copy(rows_v, acc_s.at[dst_v.at[j]], add=True)

    plsc.subcore_barrier()

    @pl.when(sid < _NS - 1)
    def _():
        pltpu.sync_copy(acc_s.at[pl.ds(row0, _RQ)],
                        out_hbm.at[cid, pl.ds(row0, _RQ)])

    @pl.when(sid == _NS - 1)
    def _():
        pltpu.sync_copy(acc_s.at[pl.ds(row0, _RL)],
                        out_hbm.at[cid, pl.ds(row0, _RL)])


def _mm_first_block(h_ref, a_ref, w_ref, o_ref):
    o_ref[...] = a_ref[...] * jnp.dot(h_ref[...], w_ref[...],
                                      preferred_element_type=jnp.float32)


def _mm_mid_block(p_ref, g_ref, a_ref, b_ref, bias_ref, w_ref, o_ref):
    t = b_ref[...] * (p_ref[0] + p_ref[1] + g_ref[...]) + bias_ref[...]
    o_ref[...] = a_ref[...] * jnp.dot(t, w_ref[...],
                                      preferred_element_type=jnp.float32)


def _final_block(p_ref, g_ref, b_ref, bias_ref, o_ref):
    o_ref[...] = (b_ref[...] * (p_ref[0] + p_ref[1] + g_ref[...])
                  + bias_ref[...])


_f32 = jnp.float32
_out_nd = jax.ShapeDtypeStruct((_N, _D), _f32)


def kernel(inputs, edge_index, edge_weight, Ws, bs):
    src = edge_index[0]
    dst = edge_index[1]
    w = jnp.where(src == dst, jnp.zeros_like(edge_weight), edge_weight)
    deg_out = jax.ops.segment_sum(w, src, num_segments=_N) + 1.0
    deg_in = jax.ops.segment_sum(w, dst, num_segments=_N) + 1.0
    a = lax.rsqrt(deg_out)[:, None]
    b = lax.rsqrt(deg_in)[:, None]
    bias = bs[:, None, :]

    src3 = src.reshape(_NW, _C, _B)
    dst3 = dst.reshape(_NW, _C, _B)
    w3 = w.reshape(_NW, _C, _B)
    zeros = jnp.zeros((_N, _D), _f32)

    g = pl.pallas_call(_mm_first_block, out_shape=_out_nd)(inputs, a, Ws[0])
    for l in range(3):
        parts = _sc_edge_pass(g, src3, dst3, w3, zeros)
        if l < 2:
            g = pl.pallas_call(_mm_mid_block, out_shape=_out_nd)(
                parts, g, a, b, bias[l], Ws[l + 1])
        else:
            h = pl.pallas_call(_final_block, out_shape=_out_nd)(
                parts, g, b, bias[l])
    return h
